# baseline (speedup 1.0000x reference)
import functools

import jax
import jax.numpy as jnp
from jax import lax
from jax.experimental import pallas as pl
from jax.experimental.pallas import tpu as pltpu
from jax.experimental.pallas import tpu_sc as plsc

_NC = 2
_NS = 16
_NW = _NC * _NS
_L = 16


@functools.lru_cache(maxsize=None)
def _make_gather(S, B, D, dtype_name):
    dtype = jnp.dtype(dtype_name)
    BB = B // _NW
    mesh = plsc.VectorSubcoreMesh(core_axis_name="c", subcore_axis_name="s")

    def body(idx_hbm, table_hbm, out_hbm, idx_v, widx2, wbuf2, tbuf2, rottab,
             gsem, wsem):
        wid = lax.axis_index("s") * _NC + lax.axis_index("c")
        b0 = wid * BB
        pltpu.sync_copy(idx_hbm.at[:, pl.ds(b0, BB)], idx_v)

        def comp_widx(s, h):
            for g in range(BB // _L):
                iv = idx_v[s, pl.ds(g * _L, _L)]
                widx2[h, pl.ds(g * _L, _L)] = lax.shift_right_logical(iv, 1)

        def gdesc(h):
            return pltpu.make_async_copy(
                table_hbm.at[widx2.at[h]], wbuf2.at[h], gsem)

        def wdesc(s, h):
            return pltpu.make_async_copy(
                tbuf2.at[h], out_hbm.at[s, :, pl.ds(b0, BB)], wsem)

        def fill_rottab():
            iota = lax.iota(jnp.int32, _L)
            for k in range(_L):
                rottab[k, :] = (iota + jnp.int32(k)) & jnp.int32(_L - 1)

        def transpose(s, h):
            iota = lax.iota(jnp.int32, _L)
            for g in range(BB // _L):
                iv = idx_v[s, pl.ds(g * _L, _L)]
                cb = (iv & jnp.int32(1)) * jnp.int32(D)
                rowv = iota + jnp.int32(g * _L)
                for d0 in range(0, D, _L):
                    cbd0 = cb + jnp.int32(d0)
                    for k in range(_L):
                        rot = rottab[k, :]
                        v = plsc.load_gather(wbuf2.at[h], [rowv, cbd0 + rot])
                        plsc.store_scatter(
                            tbuf2.at[h], [rot + jnp.int32(d0), rowv], v)

        fill_rottab()
        comp_widx(0, 0)
        gdesc(0).start()

        @pl.loop(0, S, step=2)
        def _(s0):
            for h in range(2):
                s = s0 + h
                gdesc(h).wait()

                @pl.when(s + 1 < S)
                def _():
                    comp_widx(s + 1, 1 - h)
                    gdesc(1 - h).start()

                @pl.when(s >= 2)
                def _():
                    wdesc(0, h).wait()

                transpose(s, h)
                wdesc(s, h).start()

        for h in range(2):
            wdesc(0, h).wait()

    return pl.kernel(
        body,
        out_type=jax.ShapeDtypeStruct((S, D, B), dtype),
        mesh=mesh,
        compiler_params=pltpu.CompilerParams(
            use_tc_tiling_on_sc=True, needs_layout_passes=False),
        scratch_types=[
            pltpu.VMEM((S, BB), jnp.int32),
            pltpu.VMEM((2, BB), jnp.int32),
            pltpu.VMEM((2, BB, 2 * D), dtype),
            pltpu.VMEM((2, D, BB), dtype),
            pltpu.VMEM((_L, _L), jnp.int32),
            pltpu.SemaphoreType.DMA,
            pltpu.SemaphoreType.DMA,
        ],
    )


def kernel(indices, table):
    B, S = indices.shape
    V, D = table.shape
    table2 = table.reshape(V // 2, 2 * D)
    idx_t = indices.astype(jnp.int32).T
    out_k = _make_gather(S, B, D, table.dtype.name)(idx_t, table2)
    return jnp.transpose(out_k, (2, 0, 1))


# speedup vs baseline: 1.6204x; 1.6204x over previous
import functools

import jax
import jax.numpy as jnp
from jax import lax
from jax.experimental import pallas as pl
from jax.experimental.pallas import tpu as pltpu
from jax.experimental.pallas import tpu_sc as plsc

_NC = 2
_NS = 16
_NW = _NC * _NS

_CHUNK = 512
_GROUP = 1


@functools.lru_cache(maxsize=None)
def _make_gather(N, D, dtype_name):
    dtype = jnp.dtype(dtype_name)
    n_per_w = N // _NW
    n_chunks = n_per_w // _CHUNK
    mesh = plsc.VectorSubcoreMesh(core_axis_name="c", subcore_axis_name="s")

    K = _GROUP
    n_groups = n_chunks // K

    def body(idx_hbm, table_hbm, out_hbm, idx_v, bufs, gsems, wsems):
        wid = lax.axis_index("s") * _NC + lax.axis_index("c")
        base = wid * n_per_w
        pltpu.sync_copy(idx_hbm.at[wid], idx_v)

        def gather(j, h, b):
            return pltpu.make_async_copy(
                table_hbm.at[idx_v.at[j]], bufs.at[h, b], gsems.at[h])

        def write(j, h, b):
            return pltpu.make_async_copy(
                bufs.at[h, b], out_hbm.at[pl.ds(base + j * _CHUNK, _CHUNK)],
                wsems.at[h])

        @pl.loop(0, n_groups, step=2)
        def _(g0):
            for h in range(2):
                g = g0 + h
                j0 = g * K

                @pl.when(g >= 2)
                def _():
                    for b in range(K):
                        write(0, h, b).wait()

                for b in range(K):
                    gather(j0 + b, h, b).start()
                for b in range(K):
                    gather(j0 + b, h, b).wait()
                for b in range(K):
                    write(j0 + b, h, b).start()

        for h in range(2):
            for b in range(K):
                write(0, h, b).wait()

    return pl.kernel(
        body,
        out_type=jax.ShapeDtypeStruct((N, D), dtype),
        mesh=mesh,
        compiler_params=pltpu.CompilerParams(use_tc_tiling_on_sc=False),
        scratch_types=[
            pltpu.VMEM((n_chunks, _CHUNK), jnp.int32),
            pltpu.VMEM((2, K, _CHUNK, D), dtype),
            pltpu.SemaphoreType.DMA((2,)),
            pltpu.SemaphoreType.DMA((2,)),
        ],
    )


def kernel(indices, table):
    B, S = indices.shape
    V, D = table.shape
    N = B * S
    idx = indices.astype(jnp.int32).reshape(_NW, -1, _CHUNK)
    out = _make_gather(N, D, table.dtype.name)(idx, table)
    return out.reshape(B, S, D)
